# baseline (speedup 1.0000x reference)
import functools

import jax
import jax.numpy as jnp
from jax import lax
from jax.experimental import pallas as pl
from jax.experimental.pallas import tpu as pltpu
from jax.experimental.pallas import tpu_sc as plsc

N = 10000
NP = 10240
E = 320000
C = 128
K = 80
KH = K // 2
EP = 32 * K * C
NC = 2
NS = 16
RPT = NP // NS
DEGW = 16
RB = 1024
_F32 = jnp.float32

_MESH = plsc.VectorSubcoreMesh(core_axis_name="c", subcore_axis_name="s")


def _deg_body(dstr_hbm, out_hbm, didx, ones_v, zbuf, acc):
    c = lax.axis_index("c")
    s = lax.axis_index("s")
    w = c * NS + s
    for r in range(16):
        zbuf[r] = jnp.zeros((16,), _F32)
    for r in range(C):
        ones_v[r] = jnp.ones((16,), _F32)
    for j in range(RPT // 16):
        pltpu.sync_copy(zbuf, acc.at[pl.ds(s * RPT + j * 16, 16)])
    pltpu.sync_copy(dstr_hbm.at[pl.ds(w * K, K)], didx)
    plsc.subcore_barrier()
    for k in range(K):
        pltpu.sync_copy(ones_v, acc.at[didx.at[k]], add=True)
    plsc.subcore_barrier()
    for j in range(RPT // C):
        b = s * RPT + j * C
        pltpu.sync_copy(acc.at[pl.ds(b, C)], out_hbm.at[c, pl.ds(b, C)])


_deg_call = pl.kernel(
    _deg_body,
    out_type=jax.ShapeDtypeStruct((NC, NP, DEGW), _F32),
    mesh=_MESH,
    scratch_types=[
        pltpu.VMEM((K, C), jnp.int32),
        pltpu.VMEM((C, DEGW), _F32),
        pltpu.VMEM((16, DEGW), _F32),
        pltpu.VMEM_SHARED((NP, DEGW), _F32),
    ],
)


def _agg_body(g_hbm, srcr_hbm, dstr_hbm, out_hbm, sidx, didx, rows, zbuf, acc,
              sems, ssems, *, d):
    c = lax.axis_index("c")
    s = lax.axis_index("s")
    w = c * NS + s
    for r in range(16):
        for j in range(d // 16):
            zbuf[r, pl.ds(j * 16, 16)] = jnp.zeros((16,), _F32)
    for j in range(RPT // 16):
        pltpu.sync_copy(zbuf, acc.at[pl.ds(s * RPT + j * 16, 16)])

    def load_idx(h):
        pltpu.sync_copy(srcr_hbm.at[pl.ds(w * K + h * KH, KH)], sidx)
        pltpu.sync_copy(dstr_hbm.at[pl.ds(w * K + h * KH, KH)], didx)

    load_idx(0)
    plsc.subcore_barrier()

    def gather(kk, slot):
        return pltpu.async_copy(g_hbm.at[sidx.at[kk]], rows.at[slot],
                                sems.at[slot])

    def scat(kk, slot):
        return pltpu.async_copy(rows.at[slot], acc.at[didx.at[kk]],
                                ssems.at[slot], add=True)

    descs_g = [gather(0, 0), None]
    descs_s = [None, None]
    for k in range(K):
        slot = k & 1
        boundary = (k + 1) % KH == 0 and k + 1 < K
        if k + 1 < K and not boundary:
            if descs_s[1 - slot] is not None:
                descs_s[1 - slot].wait()
            descs_g[1 - slot] = gather((k + 1) % KH, 1 - slot)
        descs_g[slot].wait()
        descs_s[slot] = scat(k % KH, slot)
        if boundary:
            descs_s[slot].wait()
            if descs_s[1 - slot] is not None:
                descs_s[1 - slot].wait()
            descs_s = [None, None]
            load_idx((k + 1) // KH)
            descs_g[1 - slot] = gather(0, 1 - slot)
    for d in descs_s:
        if d is not None:
            d.wait()
    plsc.subcore_barrier()
    wdescs = []
    for j in range(RPT // C):
        b = s * RPT + j * C
        wdescs.append(pltpu.async_copy(acc.at[pl.ds(b, C)],
                                       out_hbm.at[c, pl.ds(b, C)],
                                       ssems.at[0]))
    for d in wdescs:
        d.wait()


def _make_agg(d):
    return pl.kernel(
        functools.partial(_agg_body, d=d),
        out_type=jax.ShapeDtypeStruct((NC, NP, d), _F32),
        mesh=_MESH,
        scratch_types=[
            pltpu.VMEM((KH, C), jnp.int32),
            pltpu.VMEM((KH, C), jnp.int32),
            pltpu.VMEM((2, C, d), _F32),
            pltpu.VMEM((16, d), _F32),
            pltpu.VMEM_SHARED((NP, d), _F32),
            pltpu.SemaphoreType.DMA((2,)),
            pltpu.SemaphoreType.DMA((2,)),
        ],
    )


_agg128 = _make_agg(128)


def _tc1_body(x_ref, w_ref, dp_ref, g_ref, dv_ref):
    deg = dp_ref[0, :, 0:1] + dp_ref[1, :, 0:1] + 1.0
    dinv = lax.rsqrt(deg)
    dv_ref[...] = dinv
    h = jnp.dot(x_ref[...], w_ref[...], preferred_element_type=_F32)
    g_ref[...] = h * dinv


def _tc1(x, w1, degp):
    return pl.pallas_call(
        _tc1_body,
        grid=(NP // RB,),
        in_specs=[
            pl.BlockSpec((RB, 128), lambda i: (i, 0)),
            pl.BlockSpec((128, 128), lambda i: (0, 0)),
            pl.BlockSpec((NC, RB, DEGW), lambda i: (0, i, 0)),
        ],
        out_specs=[
            pl.BlockSpec((RB, 128), lambda i: (i, 0)),
            pl.BlockSpec((RB, 1), lambda i: (i, 0)),
        ],
        out_shape=[
            jax.ShapeDtypeStruct((NP, 128), _F32),
            jax.ShapeDtypeStruct((NP, 1), _F32),
        ],
    )(x, w1, degp)


def _tc2_body(p_ref, g_ref, dv_ref, b_ref, w_ref, o_ref):
    dinv = dv_ref[...]
    o = dinv * (p_ref[0] + p_ref[1] + g_ref[...]) + b_ref[...]
    h = jnp.maximum(o, 0.0)
    o_ref[...] = jnp.dot(h, w_ref[...], preferred_element_type=_F32) * dinv


def _tc2(p1, g1, dinv, b1, w2):
    return pl.pallas_call(
        _tc2_body,
        grid=(NP // RB,),
        in_specs=[
            pl.BlockSpec((NC, RB, 128), lambda i: (0, i, 0)),
            pl.BlockSpec((RB, 128), lambda i: (i, 0)),
            pl.BlockSpec((RB, 1), lambda i: (i, 0)),
            pl.BlockSpec((1, 128), lambda i: (0, 0)),
            pl.BlockSpec((128, 128), lambda i: (0, 0)),
        ],
        out_specs=pl.BlockSpec((RB, 128), lambda i: (i, 0)),
        out_shape=jax.ShapeDtypeStruct((NP, 128), _F32),
    )(p1, g1, dinv, b1, w2)


def _tc3_body(p_ref, g_ref, dv_ref, b_ref, o_ref):
    o = dv_ref[...] * (p_ref[0] + p_ref[1] + g_ref[...]) + b_ref[...]
    valid = lax.broadcasted_iota(jnp.int32, (RB, 128), 1) < 64
    m = jnp.max(jnp.where(valid, o, -1e30), axis=-1, keepdims=True)
    e = o - m
    lse = jnp.log(jnp.sum(jnp.where(valid, jnp.exp(e), 0.0), axis=-1,
                          keepdims=True))
    o_ref[...] = e - lse


def _tc3(p2, g2, dinv, b2):
    return pl.pallas_call(
        _tc3_body,
        grid=(NP // RB,),
        in_specs=[
            pl.BlockSpec((NC, RB, 128), lambda i: (0, i, 0)),
            pl.BlockSpec((RB, 128), lambda i: (i, 0)),
            pl.BlockSpec((RB, 1), lambda i: (i, 0)),
            pl.BlockSpec((1, 128), lambda i: (0, 0)),
        ],
        out_specs=pl.BlockSpec((RB, 128), lambda i: (i, 0)),
        out_shape=jax.ShapeDtypeStruct((NP, 128), _F32),
    )(p2, g2, dinv, b2)


def kernel(data, edge_index, W1, b1, W2, b2):
    src = edge_index[0]
    dst = edge_index[1]
    pad = jnp.full((EP - E,), N, jnp.int32)
    srcr = jnp.concatenate([src, pad]).reshape(EP // C, C)
    dstr = jnp.concatenate([dst, pad]).reshape(EP // C, C)
    x = jnp.zeros((NP, 128), _F32).at[:N].set(data)

    w2p = jnp.zeros((128, 128), _F32).at[:, :64].set(W2)
    b2p = jnp.zeros((1, 128), _F32).at[0, :64].set(b2)

    degp = _deg_call(dstr)
    g1, dinv = _tc1(x, W1, degp)
    p1 = _agg128(g1, srcr, dstr)
    g2 = _tc2(p1, g1, dinv, b1.reshape(1, 128), w2p)
    p2 = _agg128(g2, srcr, dstr)
    out = _tc3(p2, g2, dinv, b2p)
    return out[:N, :64]


# speedup vs baseline: 1.0274x; 1.0274x over previous
import functools

import jax
import jax.numpy as jnp
from jax import lax
from jax.experimental import pallas as pl
from jax.experimental.pallas import tpu as pltpu
from jax.experimental.pallas import tpu_sc as plsc

N = 10000
NP = 10240
E = 320000
C = 128
K0 = 120
K1 = 40
KH = 40
NCH = 16 * (K0 + K1)
EP = NCH * C
NC = 2
NS = 16
RPT = NP // NS
DEGW = 16
RB = 1024
_F32 = jnp.float32

_MESH = plsc.VectorSubcoreMesh(core_axis_name="c", subcore_axis_name="s")


_KD = NCH // 32


def _deg_body(dstr_hbm, out_hbm, didx, ones_v, zbuf, acc):
    c = lax.axis_index("c")
    s = lax.axis_index("s")
    w = c * NS + s
    for r in range(16):
        zbuf[r] = jnp.zeros((16,), _F32)
    for r in range(C):
        ones_v[r] = jnp.ones((16,), _F32)
    for j in range(RPT // 16):
        pltpu.sync_copy(zbuf, acc.at[pl.ds(s * RPT + j * 16, 16)])
    pltpu.sync_copy(dstr_hbm.at[pl.ds(w * _KD, _KD)], didx)
    plsc.subcore_barrier()
    for k in range(_KD):
        pltpu.sync_copy(ones_v, acc.at[didx.at[k]], add=True)
    plsc.subcore_barrier()
    for j in range(RPT // C):
        b = s * RPT + j * C
        pltpu.sync_copy(acc.at[pl.ds(b, C)], out_hbm.at[c, pl.ds(b, C)])


_deg_call = pl.kernel(
    _deg_body,
    out_type=jax.ShapeDtypeStruct((NC, NP, DEGW), _F32),
    mesh=_MESH,
    scratch_types=[
        pltpu.VMEM((_KD, C), jnp.int32),
        pltpu.VMEM((C, DEGW), _F32),
        pltpu.VMEM((16, DEGW), _F32),
        pltpu.VMEM_SHARED((NP, DEGW), _F32),
    ],
)


def _agg_body(g_hbm, srcr_hbm, dstr_hbm, out_hbm, sidx, didx, rows, zbuf, acc,
              sems, ssems, *, d):
    c = lax.axis_index("c")
    s = lax.axis_index("s")
    for r in range(16):
        for j in range(d // 16):
            zbuf[r, pl.ds(j * 16, 16)] = jnp.zeros((16,), _F32)
    for j in range(RPT // 16):
        pltpu.sync_copy(zbuf, acc.at[pl.ds(s * RPT + j * 16, 16)])

    def gather(kk, slot):
        return pltpu.async_copy(g_hbm.at[sidx.at[kk]], rows.at[slot],
                                sems.at[slot])

    def scat(kk, slot):
        return pltpu.async_copy(rows.at[slot], acc.at[didx.at[kk]],
                                ssems.at[slot], add=True)

    def edge_loop(base_row, kc):
        def load_idx(h):
            pltpu.sync_copy(srcr_hbm.at[pl.ds(base_row + h * KH, KH)], sidx)
            pltpu.sync_copy(dstr_hbm.at[pl.ds(base_row + h * KH, KH)], didx)

        load_idx(0)
        descs_g = [gather(0, 0), None]
        descs_s = [None, None]
        for k in range(kc):
            slot = k & 1
            boundary = (k + 1) % KH == 0 and k + 1 < kc
            if k + 1 < kc and not boundary:
                if descs_s[1 - slot] is not None:
                    descs_s[1 - slot].wait()
                descs_g[1 - slot] = gather((k + 1) % KH, 1 - slot)
            descs_g[slot].wait()
            descs_s[slot] = scat(k % KH, slot)
            if boundary:
                descs_s[slot].wait()
                if descs_s[1 - slot] is not None:
                    descs_s[1 - slot].wait()
                descs_s = [None, None]
                load_idx((k + 1) // KH)
                descs_g[1 - slot] = gather(0, 1 - slot)
        for ds_ in descs_s:
            if ds_ is not None:
                ds_.wait()

    plsc.subcore_barrier()

    @pl.when(c == 0)
    def _():
        edge_loop(s * K0, K0)

    @pl.when(c == 1)
    def _():
        edge_loop(16 * K0 + s * K1, K1)

    plsc.subcore_barrier()
    wdescs = []
    for j in range(RPT // C):
        b = s * RPT + j * C
        wdescs.append(pltpu.async_copy(acc.at[pl.ds(b, C)],
                                       out_hbm.at[c, pl.ds(b, C)],
                                       ssems.at[0]))
    for d in wdescs:
        d.wait()


def _make_agg(d):
    return pl.kernel(
        functools.partial(_agg_body, d=d),
        out_type=jax.ShapeDtypeStruct((NC, NP, d), _F32),
        mesh=_MESH,
        scratch_types=[
            pltpu.VMEM((KH, C), jnp.int32),
            pltpu.VMEM((KH, C), jnp.int32),
            pltpu.VMEM((2, C, d), _F32),
            pltpu.VMEM((16, d), _F32),
            pltpu.VMEM_SHARED((NP, d), _F32),
            pltpu.SemaphoreType.DMA((2,)),
            pltpu.SemaphoreType.DMA((2,)),
        ],
    )


_agg128 = _make_agg(128)


def _tc1_body(x_ref, w_ref, dp_ref, g_ref, dv_ref):
    deg = dp_ref[0, :, 0:1] + dp_ref[1, :, 0:1] + 1.0
    dinv = lax.rsqrt(deg)
    dv_ref[...] = dinv
    h = jnp.dot(x_ref[...], w_ref[...], preferred_element_type=_F32)
    g_ref[...] = h * dinv


def _tc1(x, w1, degp):
    return pl.pallas_call(
        _tc1_body,
        grid=(NP // RB,),
        in_specs=[
            pl.BlockSpec((RB, 128), lambda i: (i, 0)),
            pl.BlockSpec((128, 128), lambda i: (0, 0)),
            pl.BlockSpec((NC, RB, DEGW), lambda i: (0, i, 0)),
        ],
        out_specs=[
            pl.BlockSpec((RB, 128), lambda i: (i, 0)),
            pl.BlockSpec((RB, 1), lambda i: (i, 0)),
        ],
        out_shape=[
            jax.ShapeDtypeStruct((NP, 128), _F32),
            jax.ShapeDtypeStruct((NP, 1), _F32),
        ],
    )(x, w1, degp)


def _tc2_body(p_ref, g_ref, dv_ref, b_ref, w_ref, o_ref):
    dinv = dv_ref[...]
    o = dinv * (p_ref[0] + p_ref[1] + g_ref[...]) + b_ref[...]
    h = jnp.maximum(o, 0.0)
    o_ref[...] = jnp.dot(h, w_ref[...], preferred_element_type=_F32) * dinv


def _tc2(p1, g1, dinv, b1, w2):
    return pl.pallas_call(
        _tc2_body,
        grid=(NP // RB,),
        in_specs=[
            pl.BlockSpec((NC, RB, 128), lambda i: (0, i, 0)),
            pl.BlockSpec((RB, 128), lambda i: (i, 0)),
            pl.BlockSpec((RB, 1), lambda i: (i, 0)),
            pl.BlockSpec((1, 128), lambda i: (0, 0)),
            pl.BlockSpec((128, 128), lambda i: (0, 0)),
        ],
        out_specs=pl.BlockSpec((RB, 128), lambda i: (i, 0)),
        out_shape=jax.ShapeDtypeStruct((NP, 128), _F32),
    )(p1, g1, dinv, b1, w2)


def _tc3_body(p_ref, g_ref, dv_ref, b_ref, o_ref):
    o = dv_ref[...] * (p_ref[0] + p_ref[1] + g_ref[...]) + b_ref[...]
    valid = lax.broadcasted_iota(jnp.int32, (RB, 128), 1) < 64
    m = jnp.max(jnp.where(valid, o, -1e30), axis=-1, keepdims=True)
    e = o - m
    lse = jnp.log(jnp.sum(jnp.where(valid, jnp.exp(e), 0.0), axis=-1,
                          keepdims=True))
    o_ref[...] = e - lse


def _tc3(p2, g2, dinv, b2):
    return pl.pallas_call(
        _tc3_body,
        grid=(NP // RB,),
        in_specs=[
            pl.BlockSpec((NC, RB, 128), lambda i: (0, i, 0)),
            pl.BlockSpec((RB, 128), lambda i: (i, 0)),
            pl.BlockSpec((RB, 1), lambda i: (i, 0)),
            pl.BlockSpec((1, 128), lambda i: (0, 0)),
        ],
        out_specs=pl.BlockSpec((RB, 128), lambda i: (i, 0)),
        out_shape=jax.ShapeDtypeStruct((NP, 128), _F32),
    )(p2, g2, dinv, b2)


def kernel(data, edge_index, W1, b1, W2, b2):
    src = edge_index[0]
    dst = edge_index[1]
    pad = jnp.full((EP - E,), N, jnp.int32)
    srcr = jnp.concatenate([src, pad]).reshape(EP // C, C)
    dstr = jnp.concatenate([dst, pad]).reshape(EP // C, C)
    x = jnp.zeros((NP, 128), _F32).at[:N].set(data)

    w2p = jnp.zeros((128, 128), _F32).at[:, :64].set(W2)
    b2p = jnp.zeros((1, 128), _F32).at[0, :64].set(b2)

    degp = _deg_call(dstr)
    g1, dinv = _tc1(x, W1, degp)
    p1 = _agg128(g1, srcr, dstr)
    g2 = _tc2(p1, g1, dinv, b1.reshape(1, 128), w2p)
    p2 = _agg128(g2, srcr, dstr)
    out = _tc3(p2, g2, dinv, b2p)
    return out[:N, :64]
